# baseline (speedup 1.0000x reference)
import functools

import jax
import jax.numpy as jnp
from jax import lax
from jax.experimental import pallas as pl
from jax.experimental.pallas import tpu as pltpu
from jax.experimental.pallas import tpu_sc as plsc

TMAX = 1000
TPAD = 1024
D = 128
B = 16384

NC = 2
NS = 16
NW = NC * NS
BPW = B // NW
CHUNKS = ((0, 32), (32, 96), (128, 128), (256, 128), (384, 128))


def _table_body(emb_ref, w_ref, b_ref, out_ref):
    h = jax.nn.silu(emb_ref[...])
    out_ref[pl.ds(0, TMAX), :] = lax.dot_general(
        h, w_ref[...], (((1,), (1,)), ((), ())),
        preferred_element_type=jnp.float32) + b_ref[...]


_table_call = pl.pallas_call(
    _table_body,
    out_shape=jax.ShapeDtypeStruct((TPAD, D), jnp.float32),
)


_mesh = plsc.VectorSubcoreMesh(core_axis_name="c", subcore_axis_name="s")


@functools.partial(
    pl.kernel,
    mesh=_mesh,
    out_type=jax.ShapeDtypeStruct((B, D), jnp.float32),
    scratch_types=[
        pltpu.VMEM((BPW,), jnp.int32),
        pltpu.VMEM((BPW, D), jnp.float32),
        pltpu.VMEM_SHARED((TPAD, D), jnp.float32),
        pltpu.SemaphoreType.DMA,
        pltpu.SemaphoreType.DMA,
        pltpu.SemaphoreType.DMA,
    ],
)
def _gather_call(table_hbm, idx_hbm, out_hbm, idx_v, rows_v, table_sp,
                 ssem, sem, wsem):
    sid = lax.axis_index("s")
    wid = sid * NC + lax.axis_index("c")
    slab = TPAD // NS
    slab_cp = pltpu.async_copy(
        table_hbm.at[pl.ds(sid * slab, slab)],
        table_sp.at[pl.ds(sid * slab, slab)], ssem)
    pltpu.sync_copy(idx_hbm.at[wid], idx_v)
    off0, sz0 = CHUNKS[0]
    copies = [pltpu.async_copy(
        table_hbm.at[idx_v.at[pl.ds(off0, sz0)]],
        rows_v.at[pl.ds(off0, sz0)], sem)]
    slab_cp.wait()
    plsc.subcore_barrier()
    for off, sz in CHUNKS[1:]:
        copies.append(pltpu.async_copy(
            table_sp.at[idx_v.at[pl.ds(off, sz)]],
            rows_v.at[pl.ds(off, sz)], sem))
    writes = []
    for c, (off, sz) in zip(copies, CHUNKS):
        c.wait()
        writes.append(pltpu.async_copy(
            rows_v.at[pl.ds(off, sz)],
            out_hbm.at[pl.ds(wid * BPW + off, sz)], wsem))
    for w in writes:
        w.wait()


def kernel(_t, sinu_emb, W, b):
    table = _table_call(sinu_emb, W, b.reshape(1, D))
    idx = _t.astype(jnp.int32).reshape(NW, BPW)
    return _gather_call(table, idx)


# speedup vs baseline: 1.0094x; 1.0094x over previous
import functools

import jax
import jax.numpy as jnp
from jax import lax
from jax.experimental import pallas as pl
from jax.experimental.pallas import tpu as pltpu
from jax.experimental.pallas import tpu_sc as plsc

TMAX = 1000
TPAD = 1024
D = 128
B = 16384

NC = 2
NS = 16
NW = NC * NS
BPW = B // NW
CH = 128
NCH = BPW // CH


def _table_body(emb_ref, w_ref, b_ref, out_ref):
    h = jax.nn.silu(emb_ref[...])
    out_ref[pl.ds(0, TMAX), :] = lax.dot_general(
        h, w_ref[...], (((1,), (1,)), ((), ())),
        preferred_element_type=jnp.float32) + b_ref[...]


_table_call = pl.pallas_call(
    _table_body,
    out_shape=jax.ShapeDtypeStruct((TPAD, D), jnp.float32),
)


_mesh = plsc.VectorSubcoreMesh(core_axis_name="c", subcore_axis_name="s")


@functools.partial(
    pl.kernel,
    mesh=_mesh,
    out_type=jax.ShapeDtypeStruct((B, D), jnp.float32),
    scratch_types=[
        pltpu.VMEM((NCH, CH), jnp.int32),
        pltpu.VMEM((BPW, D), jnp.float32),
        pltpu.VMEM_SHARED((TPAD, D), jnp.float32),
        pltpu.SemaphoreType.DMA,
        pltpu.SemaphoreType.DMA,
    ],
)
def _gather_call(table_hbm, idx_hbm, out_hbm, idx_v, rows_v, table_sp,
                 sem, wsem):
    sid = lax.axis_index("s")
    wid = sid * NC + lax.axis_index("c")
    slab = TPAD // NS
    pltpu.sync_copy(idx_hbm.at[pl.ds(wid * NCH, NCH)], idx_v)
    pltpu.sync_copy(table_hbm.at[pl.ds(sid * slab, slab)],
                    table_sp.at[pl.ds(sid * slab, slab)])
    plsc.subcore_barrier()
    copies = [
        pltpu.async_copy(
            table_sp.at[idx_v.at[j]], rows_v.at[pl.ds(j * CH, CH)], sem)
        for j in range(NCH)
    ]
    writes = []
    for j in range(NCH):
        copies[j].wait()
        writes.append(pltpu.async_copy(
            rows_v.at[pl.ds(j * CH, CH)],
            out_hbm.at[pl.ds(wid * BPW + j * CH, CH)], wsem))
    for w in writes:
        w.wait()


def kernel(_t, sinu_emb, W, b):
    table = _table_call(sinu_emb, W, b.reshape(1, D))
    idx = _t.astype(jnp.int32).reshape(NW * NCH, CH)
    return _gather_call(table, idx)


# speedup vs baseline: 1.0334x; 1.0238x over previous
import functools

import jax
import jax.numpy as jnp
from jax import lax
from jax.experimental import pallas as pl
from jax.experimental.pallas import tpu as pltpu
from jax.experimental.pallas import tpu_sc as plsc

TMAX = 1000
TPAD = 1024
D = 128
B = 16384

NC = 2
NS = 16
NW = NC * NS
BPW = B // NW
CH = 128
NCH = BPW // CH


def _table_body(emb_ref, w_ref, b_ref, out_ref):
    h = jax.nn.silu(emb_ref[...])
    out_ref[pl.ds(0, TMAX), :] = lax.dot_general(
        h, w_ref[...], (((1,), (1,)), ((), ())),
        preferred_element_type=jnp.float32) + b_ref[...]


_table_call = pl.pallas_call(
    _table_body,
    out_shape=jax.ShapeDtypeStruct((TPAD, D), jnp.float32),
)


_mesh = plsc.VectorSubcoreMesh(core_axis_name="c", subcore_axis_name="s")


@functools.partial(
    pl.kernel,
    mesh=_mesh,
    out_type=jax.ShapeDtypeStruct((B, D), jnp.float32),
    scratch_types=[
        pltpu.VMEM((NCH, CH), jnp.int32),
        pltpu.VMEM((BPW, D), jnp.float32),
        pltpu.VMEM_SHARED((TPAD, D), jnp.float32),
        pltpu.SemaphoreType.DMA,
        pltpu.SemaphoreType.DMA,
    ],
)
def _gather_call(table_hbm, idx_hbm, out_hbm, idx_v, rows_v, table_sp,
                 sem, wsem):
    sid = lax.axis_index("s")
    wid = sid * NC + lax.axis_index("c")
    slab = TPAD // NS
    slab_cp = pltpu.async_copy(
        table_hbm.at[pl.ds(sid * slab, slab)],
        table_sp.at[pl.ds(sid * slab, slab)], wsem)
    pltpu.sync_copy(idx_hbm.at[pl.ds(wid * NCH, NCH)], idx_v)
    slab_cp.wait()
    plsc.subcore_barrier()
    copies = [
        pltpu.async_copy(
            table_sp.at[idx_v.at[j]], rows_v.at[pl.ds(j * CH, CH)], sem)
        for j in range(NCH)
    ]
    writes = []
    for j in range(NCH):
        copies[j].wait()
        writes.append(pltpu.async_copy(
            rows_v.at[pl.ds(j * CH, CH)],
            out_hbm.at[pl.ds(wid * BPW + j * CH, CH)], wsem))
    for w in writes:
        w.wait()


def kernel(_t, sinu_emb, W, b):
    table = _table_call(sinu_emb, W, b.reshape(1, D))
    idx = _t.astype(jnp.int32).reshape(NW * NCH, CH)
    return _gather_call(table, idx)
